# Initial kernel scaffold; baseline (speedup 1.0000x reference)
import jax
import jax.numpy as jnp
from jax.experimental import pallas as pl


def kernel(x, pos, edge_index, edge_attr, params):
    raise NotImplementedError("write your pallas kernel here")


# speedup vs baseline: 3.0379x; 3.0379x over previous
import functools

import jax
import jax.numpy as jnp
from jax import lax
from jax.experimental import pallas as pl
from jax.experimental.pallas import tpu as pltpu
from jax.experimental.pallas import tpu_sc as plsc

N = 10000
E = 320000
D = 128
TAB = 144
NW = 32
EPW = E // NW
CHUNK = 80
NCHUNK = EPW // CHUNK
NPAD = 10240
STRIPE = NPAD // 16
NB = 2000
BE = 2560


def _silu(x):
    return x / (1.0 + jnp.exp(-x))


def _stats_body(pos_ref, mean_ref, std_ref):
    p = pos_ref[...]
    m = jnp.mean(p, axis=0, keepdims=True)
    var = jnp.sum((p - m) ** 2, axis=0, keepdims=True) / (N - 1)
    mean_ref[...] = m
    std_ref[...] = jnp.clip(jnp.sqrt(var), 1e-08, None)


def _stats(pos):
    return pl.pallas_call(
        _stats_body,
        out_shape=(jax.ShapeDtypeStruct((1, 3), jnp.float32),
                   jax.ShapeDtypeStruct((1, 3), jnp.float32)),
    )(pos)


def _prep_body(x_ref, pos_ref, mean_ref, std_ref, w1a_ref, w1b_ref, sel_ref,
               a_ref, b_ref, p_ref):
    posn = (pos_ref[...] - mean_ref[...]) / std_ref[...]
    a_ref[...] = jnp.dot(x_ref[...], w1a_ref[...],
                         preferred_element_type=jnp.float32)
    b_ref[...] = jnp.dot(x_ref[...], w1b_ref[...],
                         preferred_element_type=jnp.float32)
    p_ref[...] = jnp.dot(posn, sel_ref[...], preferred_element_type=jnp.float32)


def _prep(x, pos, mean, std, w1a, w1b, sel):
    full = lambda s: pl.BlockSpec(s, lambda i: (0, 0))
    return pl.pallas_call(
        _prep_body,
        grid=(N // NB,),
        in_specs=[pl.BlockSpec((NB, D), lambda i: (i, 0)),
                  pl.BlockSpec((NB, 3), lambda i: (i, 0)),
                  full((1, 3)), full((1, 3)),
                  full((D, D)), full((D, D)), full((3, 4))],
        out_specs=(pl.BlockSpec((NB, D), lambda i: (i, 0)),
                   pl.BlockSpec((NB, D), lambda i: (i, 0)),
                   pl.BlockSpec((NB, 4), lambda i: (i, 0))),
        out_shape=(jax.ShapeDtypeStruct((N, D), jnp.float32),
                   jax.ShapeDtypeStruct((N, D), jnp.float32),
                   jax.ShapeDtypeStruct((N, 4), jnp.float32)),
    )(x, pos, mean, std, w1a, w1b, sel)


def _sc_d2_body(p_hbm, src_hbm, dst_hbm, d2_hbm, ps_hbm,
                sidx, didx, d2_v, ps_v, pfl):
    cid = lax.axis_index("c")
    sid = lax.axis_index("s")
    wid = sid * 2 + cid
    li = lax.iota(jnp.int32, 16)
    lane3 = jnp.where(li == 3, 1.0, 0.0)
    zero16 = jnp.zeros((16,), jnp.float32)
    col0 = jnp.full((16,), 0, jnp.int32)

    pltpu.sync_copy(p_hbm, pfl)

    def dinit_body(r, _):
        d2_v[r, :] = zero16
        ps_v[r, :] = lane3
        return 0
    lax.fori_loop(0, CHUNK, dinit_body, 0)

    def chunk_body(ci, _):
        base = wid * EPW + ci * CHUNK
        pltpu.sync_copy(src_hbm.at[pl.ds(base, CHUNK)], sidx)
        pltpu.sync_copy(dst_hbm.at[pl.ds(base, CHUNK)], didx)

        def grp_body(g, _):
            rows = g * 16 + li
            sv4 = sidx[pl.ds(g * 16, 16)] * 4
            dv4 = didx[pl.ds(g * 16, 16)] * 4
            d2 = jnp.zeros((16,), jnp.float32)
            for c in range(3):
                psc = plsc.load_gather(pfl, [sv4 + c])
                rc = plsc.load_gather(pfl, [dv4 + c]) - psc
                d2 = d2 + rc * rc
                plsc.store_scatter(
                    ps_v, [rows, jnp.full((16,), c, jnp.int32)], psc)
            plsc.store_scatter(d2_v, [rows, col0], d2)
            return 0
        lax.fori_loop(0, CHUNK // 16, grp_body, 0)
        pltpu.sync_copy(d2_v, d2_hbm.at[pl.ds(base, CHUNK)])
        pltpu.sync_copy(ps_v, ps_hbm.at[pl.ds(base, CHUNK)])
        return 0
    lax.fori_loop(0, NCHUNK, chunk_body, 0)


def _sc_d2(pflat, src, dst):
    mesh = plsc.VectorSubcoreMesh(core_axis_name="c", subcore_axis_name="s")
    f = functools.partial(
        pl.kernel, _sc_d2_body, mesh=mesh,
        out_type=[jax.ShapeDtypeStruct((E, 16), jnp.float32),
                  jax.ShapeDtypeStruct((E, 16), jnp.float32)],
        scratch_types=[
            pltpu.VMEM((CHUNK,), jnp.int32),
            pltpu.VMEM((CHUNK,), jnp.int32),
            pltpu.VMEM((CHUNK, 16), jnp.float32),
            pltpu.VMEM((CHUNK, 16), jnp.float32),
            pltpu.VMEM((4 * N,), jnp.float32),
        ],
        compiler_params=pltpu.CompilerParams(needs_layout_passes=False),
    )()
    return f(pflat, src, dst)


def _sc_spc_body(ps_hbm, dst_hbm, spc_hbm, didx, psbuf, spc_v, zbuf, spc_sh):
    cid = lax.axis_index("c")
    sid = lax.axis_index("s")
    wid = sid * 2 + cid
    zero16 = jnp.zeros((16,), jnp.float32)

    def zrow_body(r, _):
        for j in range(8):
            zbuf[r, pl.ds(16 * j, 16)] = zero16
        return 0
    lax.fori_loop(0, 64, zrow_body, 0)

    def sinit_body(r, _):
        for j in range(8):
            spc_v[r, pl.ds(16 * j, 16)] = zero16
        return 0
    lax.fori_loop(0, CHUNK, sinit_body, 0)
    for k in range(STRIPE // 64):
        pltpu.sync_copy(zbuf, spc_sh.at[pl.ds(sid * STRIPE + k * 64, 64)])
    plsc.subcore_barrier()

    def chunk_body(ci, _):
        base = wid * EPW + ci * CHUNK
        pltpu.sync_copy(dst_hbm.at[pl.ds(base, CHUNK)], didx)
        pltpu.sync_copy(ps_hbm.at[pl.ds(base, CHUNK)], psbuf)

        def edge_body(e, _):
            spc_v[e, pl.ds(0, 16)] = psbuf[e, :]
            return 0
        lax.fori_loop(0, CHUNK, edge_body, 0)
        pltpu.sync_copy(spc_v, spc_sh.at[didx], add=True)
        return 0
    lax.fori_loop(0, NCHUNK, chunk_body, 0)

    plsc.subcore_barrier()
    pltpu.sync_copy(spc_sh.at[pl.ds(sid * STRIPE, STRIPE)],
                    spc_hbm.at[pl.ds(cid * NPAD + sid * STRIPE, STRIPE)])


def _sc_spc(ps, dst):
    mesh = plsc.VectorSubcoreMesh(core_axis_name="c", subcore_axis_name="s")
    f = functools.partial(
        pl.kernel, _sc_spc_body, mesh=mesh,
        out_type=[jax.ShapeDtypeStruct((2 * NPAD, D), jnp.float32)],
        scratch_types=[
            pltpu.VMEM((CHUNK,), jnp.int32),
            pltpu.VMEM((CHUNK, 16), jnp.float32),
            pltpu.VMEM((CHUNK, D), jnp.float32),
            pltpu.VMEM((64, D), jnp.float32),
            pltpu.VMEM_SHARED((NPAD, D), jnp.float32),
        ],
        compiler_params=pltpu.CompilerParams(needs_layout_passes=False),
    )()
    return f(ps, dst)


def _sc_gagg_body(a_hbm, b_hbm, u_hbm, src_hbm, dst_hbm, agg_hbm,
                  sidx, didx, arows, brows, urows, zbuf, agg_sh, sem1, sem2):
    cid = lax.axis_index("c")
    sid = lax.axis_index("s")
    wid = sid * 2 + cid
    zero16 = jnp.zeros((16,), jnp.float32)

    def zrow_body(r, _):
        for j in range(8):
            zbuf[r, pl.ds(16 * j, 16)] = zero16
        return 0
    lax.fori_loop(0, 64, zrow_body, 0)
    for k in range(STRIPE // 64):
        pltpu.sync_copy(zbuf, agg_sh.at[pl.ds(sid * STRIPE + k * 64, 64)])
    plsc.subcore_barrier()

    def chunk_body(ci, _):
        base = wid * EPW + ci * CHUNK
        pltpu.sync_copy(src_hbm.at[pl.ds(base, CHUNK)], sidx)
        pltpu.sync_copy(dst_hbm.at[pl.ds(base, CHUNK)], didx)
        pltpu.sync_copy(u_hbm.at[pl.ds(base, CHUNK)], urows)
        cp1 = pltpu.async_copy(a_hbm.at[didx], arows, sem1)
        cp2 = pltpu.async_copy(b_hbm.at[sidx], brows, sem2)
        cp1.wait()
        cp2.wait()

        def edge_body(e, _):
            for j in range(8):
                sl = pl.ds(16 * j, 16)
                pre = arows[e, sl] + brows[e, sl] + urows[e, sl]
                urows[e, sl] = pre / (1.0 + jnp.exp(-pre))
            return 0
        lax.fori_loop(0, CHUNK, edge_body, 0)
        pltpu.sync_copy(urows, agg_sh.at[didx], add=True)
        return 0
    lax.fori_loop(0, NCHUNK, chunk_body, 0)

    plsc.subcore_barrier()
    pltpu.sync_copy(agg_sh.at[pl.ds(sid * STRIPE, STRIPE)],
                    agg_hbm.at[pl.ds(cid * NPAD + sid * STRIPE, STRIPE)])


def _sc_gagg(aprime, bprime, u, src, dst):
    mesh = plsc.VectorSubcoreMesh(core_axis_name="c", subcore_axis_name="s")
    f = functools.partial(
        pl.kernel, _sc_gagg_body, mesh=mesh,
        out_type=[jax.ShapeDtypeStruct((2 * NPAD, D), jnp.float32)],
        scratch_types=[
            pltpu.VMEM((CHUNK,), jnp.int32),
            pltpu.VMEM((CHUNK,), jnp.int32),
            pltpu.VMEM((CHUNK, D), jnp.float32),
            pltpu.VMEM((CHUNK, D), jnp.float32),
            pltpu.VMEM((CHUNK, D), jnp.float32),
            pltpu.VMEM((64, D), jnp.float32),
            pltpu.VMEM_SHARED((NPAD, D), jnp.float32),
            pltpu.SemaphoreType.DMA,
            pltpu.SemaphoreType.DMA,
        ],
        compiler_params=pltpu.CompilerParams(needs_layout_passes=False),
    )()
    return f(aprime, bprime, u, src, dst)


def _edge_body(ea_ref, d2_ref, ew1a_ref, wd_ref, eb1_ref, ew2_ref, w1c_ref,
               eb2_ref, nb1_ref, u_ref):
    wf = jnp.dot(ew2_ref[...], w1c_ref[...], preferred_element_type=jnp.float32)
    cconst = (jnp.dot(eb2_ref[...], w1c_ref[...],
                      preferred_element_type=jnp.float32) + nb1_ref[...])
    dist = jnp.sqrt(d2_ref[:, 0:1])
    eg = (jnp.dot(ea_ref[...], ew1a_ref[...], preferred_element_type=jnp.float32)
          + dist * wd_ref[...] + eb1_ref[...])
    u_ref[...] = jnp.dot(_silu(eg), wf,
                         preferred_element_type=jnp.float32) + cconst


def _edge_mlp(ea, d2, ew1a, wd, eb1, ew2, w1c, eb2, nb1):
    full = lambda s: pl.BlockSpec(s, lambda i: (0, 0))
    return pl.pallas_call(
        _edge_body,
        grid=(E // BE,),
        in_specs=[pl.BlockSpec((BE, 16), lambda i: (i, 0)),
                  pl.BlockSpec((BE, 16), lambda i: (i, 0)),
                  full((16, D)), full((1, D)), full((1, D)),
                  full((D, D)), full((D, D)), full((1, D)), full((1, D))],
        out_specs=pl.BlockSpec((BE, D), lambda i: (i, 0)),
        out_shape=jax.ShapeDtypeStruct((E, D), jnp.float32),
    )(ea, d2, ew1a, wd, eb1, ew2, w1c, eb2, nb1)


def _fin_body(x_ref, pos_ref, mean_ref, std_ref, a0_ref, a1_ref, s0_ref,
              s1_ref, nw2_ref, nb2_ref, g_ref, bb_ref, cw1_ref, cb1_ref,
              cw2_ref, out_ref, npos_ref):
    agg = a0_ref[...] + a1_ref[...]
    spc = s0_ref[...] + s1_ref[...]
    cnt = spc[:, 3:4]
    sp = spc[:, 0:3]
    aggm = (jnp.dot(agg, nw2_ref[...], preferred_element_type=jnp.float32)
            + cnt * nb2_ref[...]) / jnp.maximum(cnt, 1.0)
    o = aggm + x_ref[...]
    mu = jnp.mean(o, axis=-1, keepdims=True)
    var = jnp.mean((o - mu) ** 2, axis=-1, keepdims=True)
    out = (o - mu) / jnp.sqrt(var + 1e-05) * g_ref[...] + bb_ref[...]
    out_ref[...] = out
    t = jnp.dot(_silu(jnp.dot(out, cw1_ref[...],
                              preferred_element_type=jnp.float32)
                      + cb1_ref[...]),
                cw2_ref[...], preferred_element_type=jnp.float32)
    posn = (pos_ref[...] - mean_ref[...]) / std_ref[...]
    npos_ref[...] = pos_ref[...] + t * (cnt * posn - sp) * std_ref[...]


def _finalize(x, pos, mean, std, a0, a1, s0, s1, nw2, nb2, g, b, cw1, cb1, cw2):
    full = lambda s: pl.BlockSpec(s, lambda i: (0, 0))
    return pl.pallas_call(
        _fin_body,
        grid=(N // NB,),
        in_specs=[pl.BlockSpec((NB, D), lambda i: (i, 0)),
                  pl.BlockSpec((NB, 3), lambda i: (i, 0)),
                  full((1, 3)), full((1, 3)),
                  pl.BlockSpec((NB, D), lambda i: (i, 0)),
                  pl.BlockSpec((NB, D), lambda i: (i, 0)),
                  pl.BlockSpec((NB, D), lambda i: (i, 0)),
                  pl.BlockSpec((NB, D), lambda i: (i, 0)),
                  full((D, D)), full((1, D)), full((1, D)), full((1, D)),
                  full((D, D)), full((1, D)), full((D, 1))],
        out_specs=(pl.BlockSpec((NB, D), lambda i: (i, 0)),
                   pl.BlockSpec((NB, 3), lambda i: (i, 0))),
        out_shape=(jax.ShapeDtypeStruct((N, D), jnp.float32),
                   jax.ShapeDtypeStruct((N, 3), jnp.float32)),
    )(x, pos, mean, std, a0, a1, s0, s1, nw2, nb2, g, b, cw1, cb1, cw2)


def kernel(x, pos, edge_index, edge_attr, params):
    src = edge_index[0]
    dst = edge_index[1]
    sel = jnp.eye(3, 4, dtype=jnp.float32)
    for p in params:
        w1a = p['node_w1'][:D]
        w1b = p['node_w1'][D:2 * D]
        w1c = p['node_w1'][2 * D:]
        mean, std = _stats(pos)
        aprime, bprime, ppack = _prep(x, pos, mean, std, w1a, w1b, sel)
        pflat = ppack.reshape(4 * N)
        d2, ps = _sc_d2(pflat, src, dst)
        (spcp,) = _sc_spc(ps, dst)
        u = _edge_mlp(edge_attr, d2,
                      p['edge_w1'][:16], p['edge_w1'][16:17],
                      p['edge_b1'][None, :], p['edge_w2'], w1c,
                      p['edge_b2'][None, :], p['node_b1'][None, :])
        (aggp,) = _sc_gagg(aprime, bprime, u, src, dst)
        x, pos = _finalize(x, pos, mean, std,
                           aggp[:N], aggp[NPAD:NPAD + N],
                           spcp[:N], spcp[NPAD:NPAD + N],
                           p['node_w2'], p['node_b2'][None, :],
                           p['ln_g'][None, :], p['ln_b'][None, :],
                           p['coord_w1'], p['coord_b1'][None, :], p['coord_w2'])
    return (x, pos)
